# baseline (speedup 1.0000x reference)
import functools

import jax
import jax.numpy as jnp
from jax import lax
from jax.experimental import pallas as pl
from jax.experimental.pallas import tpu as pltpu
from jax.experimental.pallas import tpu_sc as plsc

N = 10000
H = 128
HQ = 16
NQ = 8
E = 320000
NT = 16
NC = 2
PPC = NQ // NC
CH = 128
EPT = 20480
NCHUNK = EPT // CH
EPAD = NT * EPT
RPRE = 400
RFLUSH = 125
RPT = N // NT


def _lrelu(v):
    return jnp.where(v >= 0, v, 0.2 * v)


def _pre_body(x_ref, w_ref, as_ref, ad_ref, h4_ref, asv_ref, adv_ref):
    h = jnp.dot(x_ref[...], w_ref[...], preferred_element_type=jnp.float32)
    for q in range(NQ):
        h4_ref[q] = h[:, q * HQ:(q + 1) * HQ]
    asv_ref[...] = jnp.sum(h * as_ref[...], axis=1, keepdims=True)
    adv_ref[...] = jnp.sum(h * ad_ref[...], axis=1, keepdims=True)


def _pre(x, W, a_s, a_d):
    return pl.pallas_call(
        _pre_body,
        grid=(N // RPRE,),
        in_specs=[
            pl.BlockSpec((RPRE, H), lambda i: (i, 0)),
            pl.BlockSpec((H, H), lambda i: (0, 0)),
            pl.BlockSpec((1, H), lambda i: (0, 0)),
            pl.BlockSpec((1, H), lambda i: (0, 0)),
        ],
        out_specs=[
            pl.BlockSpec((NQ, RPRE, HQ), lambda i: (0, i, 0)),
            pl.BlockSpec((RPRE, 1), lambda i: (i, 0)),
            pl.BlockSpec((RPRE, 1), lambda i: (i, 0)),
        ],
        out_shape=[
            jax.ShapeDtypeStruct((NQ, N, HQ), jnp.float32),
            jax.ShapeDtypeStruct((N, 1), jnp.float32),
            jax.ShapeDtypeStruct((N, 1), jnp.float32),
        ],
    )(x, W, a_s.reshape(1, H), a_d.reshape(1, H))


def _mk_body(asv_ref, adv_ref, m_ref):
    m = _lrelu(jnp.max(asv_ref[...]) + jnp.max(adv_ref[...]))
    m_ref[...] = jnp.full((8, 128), m, jnp.float32)


def _mk(asv, adv):
    return pl.pallas_call(
        _mk_body,
        out_shape=jax.ShapeDtypeStruct((8, 128), jnp.float32),
    )(asv, adv)


def _sc_body(h4_hbm, srcp_hbm, dstp_hbm, asv_hbm, adv_hbm, m_hbm,
             acc_hbm, pden_hbm,
             src_t, dst_t, w_t, asv_t, adv_t, pden_t, m_t,
             rb0, rb1, rb2, rb3, zbuf, accspm,
             gsem0, gsem1, gsem2, gsem3, ssem0, ssem1, ssem2, ssem3):
    c = lax.axis_index("c")
    s = lax.axis_index("s")

    pltpu.sync_copy(m_hbm.at[0, pl.ds(0, 16)], m_t)
    pltpu.sync_copy(asv_hbm, asv_t)
    pltpu.sync_copy(adv_hbm, adv_t)
    pltpu.sync_copy(srcp_hbm.at[s], src_t)
    pltpu.sync_copy(dstp_hbm.at[s], dst_t)

    @pl.loop(0, RFLUSH)
    def _(r):
        for f in range(0, HQ, 16):
            zbuf[r, pl.ds(f, 16)] = jnp.zeros((16,), jnp.float32)

    @pl.loop(0, N, step=16)
    def _(i):
        pden_t[pl.ds(i, 16)] = jnp.zeros((16,), jnp.float32)

    m16 = m_t[...]

    @pl.loop(0, NCHUNK)
    def _(j):
        @pl.loop(0, CH, step=16)
        def _(k):
            s16 = src_t[j, pl.ds(k, 16)]
            d16 = dst_t[j, pl.ds(k, 16)]
            e = plsc.load_gather(asv_t, [s16]) + plsc.load_gather(adv_t, [d16])
            w = jnp.exp(_lrelu(e) - m16)
            g = s * EPT + j * CH + k + lax.iota(jnp.int32, 16)
            w = jnp.where(g < E, w, 0.0)
            w_t[j, pl.ds(k, 16)] = w
            plsc.addupdate_scatter(pden_t, [d16], w)

    @pl.when(c == 0)
    def _():
        pltpu.sync_copy(pden_t, pden_hbm.at[s])

    def _scale(buf, j):
        @pl.loop(0, CH, step=16)
        def _(k):
            w16 = w_t[j, pl.ds(k, 16)]
            for l in range(16):
                av = jnp.full((16,), w16[l], jnp.float32)
                for f in range(0, HQ, 16):
                    buf[k + l, pl.ds(f, 16)] = buf[k + l, pl.ds(f, 16)] * av

    for p in range(PPC):
        q = c * PPC + p
        hslab = h4_hbm.at[q]

        @pl.loop(0, RPT // RFLUSH)
        def _(k):
            pltpu.sync_copy(zbuf,
                            accspm.at[pl.ds(s * RPT + k * RFLUSH, RFLUSH)])

        plsc.subcore_barrier()

        bufs = (rb0, rb1, rb2, rb3)
        gsem = (gsem0, gsem1, gsem2, gsem3)
        ssem = (ssem0, ssem1, ssem2, ssem3)

        pltpu.async_copy(hslab.at[src_t.at[0]], bufs[0], gsem[0])
        pltpu.async_copy(hslab.at[src_t.at[1]], bufs[1], gsem[1])

        @pl.loop(2, NCHUNK + 2)
        def _(j):
            for par in range(4):
                @pl.when(lax.rem(j, 4) == par)
                def _():
                    new, old = bufs[par], bufs[(par + 2) % 4]
                    @pl.when(j >= 4)
                    def _():
                        pltpu.make_async_copy(
                            acc_hbm.at[q, pl.ds(0, CH)], new,
                            ssem[par]).wait()

                    @pl.when(j < NCHUNK)
                    def _():
                        pltpu.async_copy(hslab.at[src_t.at[j]], new,
                                         gsem[par])
                    pltpu.make_async_copy(
                        hslab.at[pl.ds(0, CH)], old,
                        gsem[(par + 2) % 4]).wait()
                    _scale(old, j - 2)
                    pltpu.async_copy(old, accspm.at[dst_t.at[j - 2]],
                                     ssem[(par + 2) % 4], add=True)

        pltpu.make_async_copy(acc_hbm.at[q, pl.ds(0, CH)], bufs[2],
                              ssem[2]).wait()
        pltpu.make_async_copy(acc_hbm.at[q, pl.ds(0, CH)], bufs[3],
                              ssem[3]).wait()

        plsc.subcore_barrier()

        @pl.loop(0, RPT // RFLUSH)
        def _(k):
            base = s * RPT + k * RFLUSH
            pltpu.sync_copy(accspm.at[pl.ds(base, RFLUSH)],
                            acc_hbm.at[q, pl.ds(base, RFLUSH)])


def _sc_edges(h4, srcp, dstp, asv, adv, m):
    mesh = plsc.VectorSubcoreMesh(core_axis_name="c", subcore_axis_name="s")
    kern = pl.kernel(
        _sc_body,
        mesh=mesh,
        compiler_params=pltpu.CompilerParams(use_tc_tiling_on_sc=False,
                                             needs_layout_passes=False),
        out_type=[
            jax.ShapeDtypeStruct((NQ, N, HQ), jnp.float32),
            jax.ShapeDtypeStruct((NT, N), jnp.float32),
        ],
        scratch_types=[
            pltpu.VMEM((NCHUNK, CH), jnp.int32),
            pltpu.VMEM((NCHUNK, CH), jnp.int32),
            pltpu.VMEM((NCHUNK, CH), jnp.float32),
            pltpu.VMEM((N,), jnp.float32),
            pltpu.VMEM((N,), jnp.float32),
            pltpu.VMEM((N,), jnp.float32),
            pltpu.VMEM((16,), jnp.float32),
            pltpu.VMEM((CH, HQ), jnp.float32),
            pltpu.VMEM((CH, HQ), jnp.float32),
            pltpu.VMEM((CH, HQ), jnp.float32),
            pltpu.VMEM((CH, HQ), jnp.float32),
            pltpu.VMEM((RFLUSH, HQ), jnp.float32),
            pltpu.VMEM_SHARED((N, HQ), jnp.float32),
            pltpu.SemaphoreType.DMA,
            pltpu.SemaphoreType.DMA,
            pltpu.SemaphoreType.DMA,
            pltpu.SemaphoreType.DMA,
            pltpu.SemaphoreType.DMA,
            pltpu.SemaphoreType.DMA,
            pltpu.SemaphoreType.DMA,
            pltpu.SemaphoreType.DMA,
        ],
    )
    return kern(h4, srcp, dstp, asv, adv, m)


def _den_body(pden_ref, den_ref):
    ones = jnp.ones((NT, 1), jnp.float32)
    den_ref[...] = lax.dot_general(pden_ref[...], ones,
                                   (((0,), (0,)), ((), ())),
                                   precision=lax.Precision.HIGHEST,
                                   preferred_element_type=jnp.float32)


def _den(pden):
    return pl.pallas_call(
        _den_body,
        out_shape=jax.ShapeDtypeStruct((N, 1), jnp.float32),
    )(pden)


def _post_body(relu, acc_ref, den_ref, h4_ref, asv_ref, adv_ref, m_ref,
               b_ref, out_ref):
    sw = jnp.exp(_lrelu(asv_ref[...] + adv_ref[...]) - m_ref[0:1, 0:1])
    den = den_ref[...] + sw
    cols = [acc_ref[q] + sw * h4_ref[q] for q in range(NQ)]
    o = jnp.concatenate(cols, axis=1) / den + b_ref[...]
    if relu:
        o = jnp.maximum(o, 0.0)
    out_ref[...] = o


def _post(acc, den, h4, asv, adv, m, b, relu):
    return pl.pallas_call(
        functools.partial(_post_body, relu),
        grid=(N // RPRE,),
        in_specs=[
            pl.BlockSpec((NQ, RPRE, HQ), lambda i: (0, i, 0)),
            pl.BlockSpec((RPRE, 1), lambda i: (i, 0)),
            pl.BlockSpec((NQ, RPRE, HQ), lambda i: (0, i, 0)),
            pl.BlockSpec((RPRE, 1), lambda i: (i, 0)),
            pl.BlockSpec((RPRE, 1), lambda i: (i, 0)),
            pl.BlockSpec((8, 128), lambda i: (0, 0)),
            pl.BlockSpec((1, H), lambda i: (0, 0)),
        ],
        out_specs=pl.BlockSpec((RPRE, H), lambda i: (i, 0)),
        out_shape=jax.ShapeDtypeStruct((N, H), jnp.float32),
    )(acc, den, h4, asv, adv, m, b.reshape(1, H))


def _gat_layer(x, srcp, dstp, W, a_s, a_d, b, relu):
    h4, asv, adv = _pre(x, W, a_s, a_d)
    m = _mk(asv, adv)
    acc, pden = _sc_edges(h4, srcp, dstp,
                          asv.reshape(N), adv.reshape(N), m)
    return _post(acc, _den(pden), h4, asv, adv, m, b, relu)


def kernel(x, edge_index, W1, a_src1, a_dst1, b1, W2, a_src2, a_dst2, b2):
    src = edge_index[0].astype(jnp.int32)
    dst = edge_index[1].astype(jnp.int32)
    srcp = jnp.pad(src, (0, EPAD - E)).reshape(NT, NCHUNK, CH)
    dstp = jnp.pad(dst, (0, EPAD - E)).reshape(NT, NCHUNK, CH)
    h = _gat_layer(x, srcp, dstp, W1, a_src1, a_dst1, b1, relu=True)
    return _gat_layer(h, srcp, dstp, W2, a_src2, a_dst2, b2, relu=False)


# speedup vs baseline: 1.1534x; 1.1534x over previous
import functools

import jax
import jax.numpy as jnp
from jax import lax
from jax.experimental import pallas as pl
from jax.experimental.pallas import tpu as pltpu
from jax.experimental.pallas import tpu_sc as plsc

N = 10000
H = 128
HQ = 32
NQ = 4
E = 320000
NT = 16
NC = 2
PPC = NQ // NC
CH = 128
EPT = 20480
NCHUNK = EPT // CH
EPAD = NT * EPT
RPRE = 400
RFLUSH = 125
RPT = N // NT


def _lrelu(v):
    return jnp.where(v >= 0, v, 0.2 * v)


def _pre_body(x_ref, w_ref, as_ref, ad_ref, h4_ref, asv_ref, adv_ref):
    h = jnp.dot(x_ref[...], w_ref[...], preferred_element_type=jnp.float32)
    for q in range(NQ):
        h4_ref[q] = h[:, q * HQ:(q + 1) * HQ]
    asv_ref[...] = jnp.sum(h * as_ref[...], axis=1, keepdims=True)
    adv_ref[...] = jnp.sum(h * ad_ref[...], axis=1, keepdims=True)


def _pre(x, W, a_s, a_d):
    return pl.pallas_call(
        _pre_body,
        grid=(N // RPRE,),
        in_specs=[
            pl.BlockSpec((RPRE, H), lambda i: (i, 0)),
            pl.BlockSpec((H, H), lambda i: (0, 0)),
            pl.BlockSpec((1, H), lambda i: (0, 0)),
            pl.BlockSpec((1, H), lambda i: (0, 0)),
        ],
        out_specs=[
            pl.BlockSpec((NQ, RPRE, HQ), lambda i: (0, i, 0)),
            pl.BlockSpec((RPRE, 1), lambda i: (i, 0)),
            pl.BlockSpec((RPRE, 1), lambda i: (i, 0)),
        ],
        out_shape=[
            jax.ShapeDtypeStruct((NQ, N, HQ), jnp.float32),
            jax.ShapeDtypeStruct((N, 1), jnp.float32),
            jax.ShapeDtypeStruct((N, 1), jnp.float32),
        ],
    )(x, W, a_s.reshape(1, H), a_d.reshape(1, H))


def _mk_body(asv_ref, adv_ref, m_ref):
    m = _lrelu(jnp.max(asv_ref[...]) + jnp.max(adv_ref[...]))
    m_ref[...] = jnp.full((8, 128), m, jnp.float32)


def _mk(asv, adv):
    return pl.pallas_call(
        _mk_body,
        out_shape=jax.ShapeDtypeStruct((8, 128), jnp.float32),
    )(asv, adv)


def _sc_body(h4_hbm, srcp_hbm, dstp_hbm, asv_hbm, adv_hbm, m_hbm,
             acc_hbm, pden_hbm,
             src_t, dst_t, w_t, asv_t, adv_t, pden_t, m_t,
             rb0, rb1, rb2, zbuf, accspm,
             sem0, sem1, sem2):
    c = lax.axis_index("c")
    s = lax.axis_index("s")

    pltpu.sync_copy(m_hbm.at[0, pl.ds(0, 16)], m_t)
    pltpu.sync_copy(asv_hbm, asv_t)
    pltpu.sync_copy(adv_hbm, adv_t)
    pltpu.sync_copy(srcp_hbm.at[s], src_t)
    pltpu.sync_copy(dstp_hbm.at[s], dst_t)

    @pl.loop(0, RFLUSH)
    def _(r):
        for f in range(0, HQ, 16):
            zbuf[r, pl.ds(f, 16)] = jnp.zeros((16,), jnp.float32)

    @pl.loop(0, N, step=16)
    def _(i):
        pden_t[pl.ds(i, 16)] = jnp.zeros((16,), jnp.float32)

    m16 = m_t[...]

    @pl.loop(0, NCHUNK)
    def _(j):
        @pl.loop(0, CH, step=16)
        def _(k):
            s16 = src_t[j, pl.ds(k, 16)]
            d16 = dst_t[j, pl.ds(k, 16)]
            e = plsc.load_gather(asv_t, [s16]) + plsc.load_gather(adv_t, [d16])
            w = jnp.exp(_lrelu(e) - m16)
            g = s * EPT + j * CH + k + lax.iota(jnp.int32, 16)
            w = jnp.where(g < E, w, 0.0)
            w_t[j, pl.ds(k, 16)] = w
            plsc.addupdate_scatter(pden_t, [d16], w)

    @pl.when(c == 0)
    def _():
        pltpu.sync_copy(pden_t, pden_hbm.at[s])

    def _scale(buf, j):
        @pl.loop(0, CH, step=16)
        def _(k):
            w16 = w_t[j, pl.ds(k, 16)]
            for l in range(16):
                av = jnp.full((16,), w16[l], jnp.float32)
                for f in range(0, HQ, 16):
                    buf[k + l, pl.ds(f, 16)] = buf[k + l, pl.ds(f, 16)] * av

    for p in range(PPC):
        q = c * PPC + p
        hslab = h4_hbm.at[q]

        @pl.loop(0, RPT // RFLUSH)
        def _(k):
            pltpu.sync_copy(zbuf,
                            accspm.at[pl.ds(s * RPT + k * RFLUSH, RFLUSH)])

        plsc.subcore_barrier()

        bufs = (rb0, rb1, rb2)
        sems = (sem0, sem1, sem2)

        pltpu.async_copy(hslab.at[src_t.at[0]], bufs[0], sems[0])
        pltpu.async_copy(hslab.at[src_t.at[1]], bufs[1], sems[1])

        @pl.loop(2, NCHUNK + 2)
        def _(j):
            for par in range(3):
                @pl.when(lax.rem(j, 3) == par)
                def _():
                    new, old = bufs[par], bufs[(par + 1) % 3]
                    @pl.when(j >= 3)
                    def _():
                        pltpu.make_async_copy(
                            acc_hbm.at[q, pl.ds(0, CH)], new,
                            sems[par]).wait()

                    @pl.when(j < NCHUNK)
                    def _():
                        pltpu.async_copy(hslab.at[src_t.at[j]], new,
                                         sems[par])
                    pltpu.make_async_copy(
                        hslab.at[pl.ds(0, CH)], old,
                        sems[(par + 1) % 3]).wait()
                    _scale(old, j - 2)
                    pltpu.async_copy(old, accspm.at[dst_t.at[j - 2]],
                                     sems[(par + 1) % 3], add=True)

        pltpu.make_async_copy(acc_hbm.at[q, pl.ds(0, CH)], bufs[0],
                              sems[0]).wait()

        plsc.subcore_barrier()

        @pl.loop(0, RPT // RFLUSH)
        def _(k):
            base = s * RPT + k * RFLUSH
            pltpu.sync_copy(accspm.at[pl.ds(base, RFLUSH)],
                            acc_hbm.at[q, pl.ds(base, RFLUSH)])


def _sc_edges(h4, srcp, dstp, asv, adv, m):
    mesh = plsc.VectorSubcoreMesh(core_axis_name="c", subcore_axis_name="s")
    kern = pl.kernel(
        _sc_body,
        mesh=mesh,
        compiler_params=pltpu.CompilerParams(use_tc_tiling_on_sc=False,
                                             needs_layout_passes=False),
        out_type=[
            jax.ShapeDtypeStruct((NQ, N, HQ), jnp.float32),
            jax.ShapeDtypeStruct((NT, N), jnp.float32),
        ],
        scratch_types=[
            pltpu.VMEM((NCHUNK, CH), jnp.int32),
            pltpu.VMEM((NCHUNK, CH), jnp.int32),
            pltpu.VMEM((NCHUNK, CH), jnp.float32),
            pltpu.VMEM((N,), jnp.float32),
            pltpu.VMEM((N,), jnp.float32),
            pltpu.VMEM((N,), jnp.float32),
            pltpu.VMEM((16,), jnp.float32),
            pltpu.VMEM((CH, HQ), jnp.float32),
            pltpu.VMEM((CH, HQ), jnp.float32),
            pltpu.VMEM((CH, HQ), jnp.float32),
            pltpu.VMEM((RFLUSH, HQ), jnp.float32),
            pltpu.VMEM_SHARED((N, HQ), jnp.float32),
            pltpu.SemaphoreType.DMA,
            pltpu.SemaphoreType.DMA,
            pltpu.SemaphoreType.DMA,
        ],
    )
    return kern(h4, srcp, dstp, asv, adv, m)


def _den_body(pden_ref, den_ref):
    ones = jnp.ones((NT, 1), jnp.float32)
    den_ref[...] = lax.dot_general(pden_ref[...], ones,
                                   (((0,), (0,)), ((), ())),
                                   precision=lax.Precision.HIGHEST,
                                   preferred_element_type=jnp.float32)


def _den(pden):
    return pl.pallas_call(
        _den_body,
        out_shape=jax.ShapeDtypeStruct((N, 1), jnp.float32),
    )(pden)


def _post_body(relu, acc_ref, den_ref, h4_ref, asv_ref, adv_ref, m_ref,
               b_ref, out_ref):
    sw = jnp.exp(_lrelu(asv_ref[...] + adv_ref[...]) - m_ref[0:1, 0:1])
    den = den_ref[...] + sw
    cols = [acc_ref[q] + sw * h4_ref[q] for q in range(NQ)]
    o = jnp.concatenate(cols, axis=1) / den + b_ref[...]
    if relu:
        o = jnp.maximum(o, 0.0)
    out_ref[...] = o


def _post(acc, den, h4, asv, adv, m, b, relu):
    return pl.pallas_call(
        functools.partial(_post_body, relu),
        grid=(N // RPRE,),
        in_specs=[
            pl.BlockSpec((NQ, RPRE, HQ), lambda i: (0, i, 0)),
            pl.BlockSpec((RPRE, 1), lambda i: (i, 0)),
            pl.BlockSpec((NQ, RPRE, HQ), lambda i: (0, i, 0)),
            pl.BlockSpec((RPRE, 1), lambda i: (i, 0)),
            pl.BlockSpec((RPRE, 1), lambda i: (i, 0)),
            pl.BlockSpec((8, 128), lambda i: (0, 0)),
            pl.BlockSpec((1, H), lambda i: (0, 0)),
        ],
        out_specs=pl.BlockSpec((RPRE, H), lambda i: (i, 0)),
        out_shape=jax.ShapeDtypeStruct((N, H), jnp.float32),
    )(acc, den, h4, asv, adv, m, b.reshape(1, H))


def _gat_layer(x, srcp, dstp, W, a_s, a_d, b, relu):
    h4, asv, adv = _pre(x, W, a_s, a_d)
    m = _mk(asv, adv)
    acc, pden = _sc_edges(h4, srcp, dstp,
                          asv.reshape(N), adv.reshape(N), m)
    return _post(acc, _den(pden), h4, asv, adv, m, b, relu)


def kernel(x, edge_index, W1, a_src1, a_dst1, b1, W2, a_src2, a_dst2, b2):
    src = edge_index[0].astype(jnp.int32)
    dst = edge_index[1].astype(jnp.int32)
    srcp = jnp.pad(src, (0, EPAD - E)).reshape(NT, NCHUNK, CH)
    dstp = jnp.pad(dst, (0, EPAD - E)).reshape(NT, NCHUNK, CH)
    h = _gat_layer(x, srcp, dstp, W1, a_src1, a_dst1, b1, relu=True)
    return _gat_layer(h, srcp, dstp, W2, a_src2, a_dst2, b2, relu=False)


# speedup vs baseline: 1.2131x; 1.0518x over previous
import functools

import jax
import jax.numpy as jnp
from jax import lax
from jax.experimental import pallas as pl
from jax.experimental.pallas import tpu as pltpu
from jax.experimental.pallas import tpu_sc as plsc

N = 10000
H = 128
HQ = 32
NQ = 4
E = 320000
NT = 16
NC = 2
PPC = NQ // NC
CH = 128
EPT = 20480
NCHUNK = EPT // CH
EPAD = NT * EPT
RPRE = 400
RFLUSH = 125
RPT = N // NT


def _lrelu(v):
    return jnp.where(v >= 0, v, 0.2 * v)


def _pre_body(x_ref, w_ref, as_ref, ad_ref, h4_ref, asv_ref, adv_ref):
    h = jnp.dot(x_ref[...], w_ref[...], preferred_element_type=jnp.float32)
    for q in range(NQ):
        h4_ref[q] = h[:, q * HQ:(q + 1) * HQ]
    asv_ref[...] = jnp.sum(h * as_ref[...], axis=1, keepdims=True)
    adv_ref[...] = jnp.sum(h * ad_ref[...], axis=1, keepdims=True)


def _pre(x, W, a_s, a_d):
    return pl.pallas_call(
        _pre_body,
        grid=(N // RPRE,),
        in_specs=[
            pl.BlockSpec((RPRE, H), lambda i: (i, 0)),
            pl.BlockSpec((H, H), lambda i: (0, 0)),
            pl.BlockSpec((1, H), lambda i: (0, 0)),
            pl.BlockSpec((1, H), lambda i: (0, 0)),
        ],
        out_specs=[
            pl.BlockSpec((NQ, RPRE, HQ), lambda i: (0, i, 0)),
            pl.BlockSpec((RPRE, 1), lambda i: (i, 0)),
            pl.BlockSpec((RPRE, 1), lambda i: (i, 0)),
        ],
        out_shape=[
            jax.ShapeDtypeStruct((NQ, N, HQ), jnp.float32),
            jax.ShapeDtypeStruct((N, 1), jnp.float32),
            jax.ShapeDtypeStruct((N, 1), jnp.float32),
        ],
    )(x, W, a_s.reshape(1, H), a_d.reshape(1, H))


def _mk_body(asv_ref, adv_ref, m_ref):
    m = _lrelu(jnp.max(asv_ref[...]) + jnp.max(adv_ref[...]))
    m_ref[...] = jnp.full((8, 128), m, jnp.float32)


def _mk(asv, adv):
    return pl.pallas_call(
        _mk_body,
        out_shape=jax.ShapeDtypeStruct((8, 128), jnp.float32),
    )(asv, adv)


def _sc_body(h4_hbm, srcp_hbm, dstp_hbm, asv_hbm, adv_hbm, m_hbm,
             acc_hbm, pden_hbm,
             src_t, dst_t, w_t, asv_t, adv_t, pden_t, m_t,
             rb0, rb1, rb2, zbuf, accspm,
             sem0, sem1, sem2):
    c = lax.axis_index("c")
    s = lax.axis_index("s")

    pltpu.sync_copy(m_hbm.at[0, pl.ds(0, 16)], m_t)
    pltpu.sync_copy(asv_hbm, asv_t)
    pltpu.sync_copy(adv_hbm, adv_t)
    pltpu.sync_copy(srcp_hbm.at[s], src_t)
    pltpu.sync_copy(dstp_hbm.at[s], dst_t)

    @pl.loop(0, RFLUSH)
    def _(r):
        for f in range(0, HQ, 16):
            zbuf[r, pl.ds(f, 16)] = jnp.zeros((16,), jnp.float32)

    @pl.loop(0, N, step=16)
    def _(i):
        pden_t[pl.ds(i, 16)] = jnp.zeros((16,), jnp.float32)

    m16 = m_t[...]

    def _scale(buf, j, compute_w):
        @pl.loop(0, CH, step=16)
        def _(k):
            if compute_w:
                s16 = src_t[j, pl.ds(k, 16)]
                d16 = dst_t[j, pl.ds(k, 16)]
                e = (plsc.load_gather(asv_t, [s16])
                     + plsc.load_gather(adv_t, [d16]))
                w16 = jnp.exp(_lrelu(e) - m16)
                g = s * EPT + j * CH + k + lax.iota(jnp.int32, 16)
                w16 = jnp.where(g < E, w16, 0.0)
                w_t[j, pl.ds(k, 16)] = w16
                plsc.addupdate_scatter(pden_t, [d16], w16)
            else:
                w16 = w_t[j, pl.ds(k, 16)]
            for l in range(16):
                av = jnp.full((16,), w16[l], jnp.float32)
                for f in range(0, HQ, 16):
                    buf[k + l, pl.ds(f, 16)] = buf[k + l, pl.ds(f, 16)] * av

    for p in range(PPC):
        q = c * PPC + p
        hslab = h4_hbm.at[q]

        @pl.loop(0, RPT // RFLUSH)
        def _(k):
            pltpu.sync_copy(zbuf,
                            accspm.at[pl.ds(s * RPT + k * RFLUSH, RFLUSH)])

        plsc.subcore_barrier()

        bufs = (rb0, rb1, rb2)
        sems = (sem0, sem1, sem2)

        pltpu.async_copy(hslab.at[src_t.at[0]], bufs[0], sems[0])
        pltpu.async_copy(hslab.at[src_t.at[1]], bufs[1], sems[1])

        @pl.loop(2, NCHUNK + 2)
        def _(j):
            for par in range(3):
                @pl.when(lax.rem(j, 3) == par)
                def _():
                    new, old = bufs[par], bufs[(par + 1) % 3]
                    @pl.when(j >= 3)
                    def _():
                        pltpu.make_async_copy(
                            acc_hbm.at[q, pl.ds(0, CH)], new,
                            sems[par]).wait()

                    @pl.when(j < NCHUNK)
                    def _():
                        pltpu.async_copy(hslab.at[src_t.at[j]], new,
                                         sems[par])
                    pltpu.make_async_copy(
                        hslab.at[pl.ds(0, CH)], old,
                        sems[(par + 1) % 3]).wait()
                    _scale(old, j - 2, compute_w=(p == 0))
                    pltpu.async_copy(old, accspm.at[dst_t.at[j - 2]],
                                     sems[(par + 1) % 3], add=True)

        pltpu.make_async_copy(acc_hbm.at[q, pl.ds(0, CH)], bufs[0],
                              sems[0]).wait()

        if p == 0:
            @pl.when(c == 0)
            def _():
                pltpu.sync_copy(pden_t, pden_hbm.at[s])

        plsc.subcore_barrier()

        @pl.loop(0, RPT // RFLUSH)
        def _(k):
            base = s * RPT + k * RFLUSH
            pltpu.sync_copy(accspm.at[pl.ds(base, RFLUSH)],
                            acc_hbm.at[q, pl.ds(base, RFLUSH)])


def _sc_edges(h4, srcp, dstp, asv, adv, m):
    mesh = plsc.VectorSubcoreMesh(core_axis_name="c", subcore_axis_name="s")
    kern = pl.kernel(
        _sc_body,
        mesh=mesh,
        compiler_params=pltpu.CompilerParams(use_tc_tiling_on_sc=False,
                                             needs_layout_passes=False),
        out_type=[
            jax.ShapeDtypeStruct((NQ, N, HQ), jnp.float32),
            jax.ShapeDtypeStruct((NT, N), jnp.float32),
        ],
        scratch_types=[
            pltpu.VMEM((NCHUNK, CH), jnp.int32),
            pltpu.VMEM((NCHUNK, CH), jnp.int32),
            pltpu.VMEM((NCHUNK, CH), jnp.float32),
            pltpu.VMEM((N,), jnp.float32),
            pltpu.VMEM((N,), jnp.float32),
            pltpu.VMEM((N,), jnp.float32),
            pltpu.VMEM((16,), jnp.float32),
            pltpu.VMEM((CH, HQ), jnp.float32),
            pltpu.VMEM((CH, HQ), jnp.float32),
            pltpu.VMEM((CH, HQ), jnp.float32),
            pltpu.VMEM((RFLUSH, HQ), jnp.float32),
            pltpu.VMEM_SHARED((N, HQ), jnp.float32),
            pltpu.SemaphoreType.DMA,
            pltpu.SemaphoreType.DMA,
            pltpu.SemaphoreType.DMA,
        ],
    )
    return kern(h4, srcp, dstp, asv, adv, m)


def _den_body(pden_ref, den_ref):
    ones = jnp.ones((NT, 1), jnp.float32)
    den_ref[...] = lax.dot_general(pden_ref[...], ones,
                                   (((0,), (0,)), ((), ())),
                                   precision=lax.Precision.HIGHEST,
                                   preferred_element_type=jnp.float32)


def _den(pden):
    return pl.pallas_call(
        _den_body,
        out_shape=jax.ShapeDtypeStruct((N, 1), jnp.float32),
    )(pden)


def _post_body(relu, acc_ref, den_ref, h4_ref, asv_ref, adv_ref, m_ref,
               b_ref, out_ref):
    sw = jnp.exp(_lrelu(asv_ref[...] + adv_ref[...]) - m_ref[0:1, 0:1])
    den = den_ref[...] + sw
    cols = [acc_ref[q] + sw * h4_ref[q] for q in range(NQ)]
    o = jnp.concatenate(cols, axis=1) / den + b_ref[...]
    if relu:
        o = jnp.maximum(o, 0.0)
    out_ref[...] = o


def _post(acc, den, h4, asv, adv, m, b, relu):
    return pl.pallas_call(
        functools.partial(_post_body, relu),
        grid=(N // RPRE,),
        in_specs=[
            pl.BlockSpec((NQ, RPRE, HQ), lambda i: (0, i, 0)),
            pl.BlockSpec((RPRE, 1), lambda i: (i, 0)),
            pl.BlockSpec((NQ, RPRE, HQ), lambda i: (0, i, 0)),
            pl.BlockSpec((RPRE, 1), lambda i: (i, 0)),
            pl.BlockSpec((RPRE, 1), lambda i: (i, 0)),
            pl.BlockSpec((8, 128), lambda i: (0, 0)),
            pl.BlockSpec((1, H), lambda i: (0, 0)),
        ],
        out_specs=pl.BlockSpec((RPRE, H), lambda i: (i, 0)),
        out_shape=jax.ShapeDtypeStruct((N, H), jnp.float32),
    )(acc, den, h4, asv, adv, m, b.reshape(1, H))


def _gat_layer(x, srcp, dstp, W, a_s, a_d, b, relu):
    h4, asv, adv = _pre(x, W, a_s, a_d)
    m = _mk(asv, adv)
    acc, pden = _sc_edges(h4, srcp, dstp,
                          asv.reshape(N), adv.reshape(N), m)
    return _post(acc, _den(pden), h4, asv, adv, m, b, relu)


def kernel(x, edge_index, W1, a_src1, a_dst1, b1, W2, a_src2, a_dst2, b2):
    src = edge_index[0].astype(jnp.int32)
    dst = edge_index[1].astype(jnp.int32)
    srcp = jnp.pad(src, (0, EPAD - E)).reshape(NT, NCHUNK, CH)
    dstp = jnp.pad(dst, (0, EPAD - E)).reshape(NT, NCHUNK, CH)
    h = _gat_layer(x, srcp, dstp, W1, a_src1, a_dst1, b1, relu=True)
    return _gat_layer(h, srcp, dstp, W2, a_src2, a_dst2, b2, relu=False)


# speedup vs baseline: 1.2153x; 1.0018x over previous
import functools

import jax
import jax.numpy as jnp
from jax import lax
from jax.experimental import pallas as pl
from jax.experimental.pallas import tpu as pltpu
from jax.experimental.pallas import tpu_sc as plsc

N = 10000
H = 128
HQ = 32
NQ = 4
E = 320000
NT = 16
NC = 2
PPC = NQ // NC
CH = 128
EPT = 20480
NCHUNK = EPT // CH
EPAD = NT * EPT
RPRE = 400
RFLUSH = 125
RPT = N // NT


def _lrelu(v):
    return jnp.where(v >= 0, v, 0.2 * v)


def _pre_body(x_ref, w_ref, as_ref, ad_ref, h4_ref, asv_ref, adv_ref):
    h = jnp.dot(x_ref[...], w_ref[...], preferred_element_type=jnp.float32)
    for q in range(NQ):
        h4_ref[q] = h[:, q * HQ:(q + 1) * HQ]
    asv_ref[...] = jnp.sum(h * as_ref[...], axis=1, keepdims=True)
    adv_ref[...] = jnp.sum(h * ad_ref[...], axis=1, keepdims=True)


def _pre(x, W, a_s, a_d):
    return pl.pallas_call(
        _pre_body,
        grid=(N // RPRE,),
        in_specs=[
            pl.BlockSpec((RPRE, H), lambda i: (i, 0)),
            pl.BlockSpec((H, H), lambda i: (0, 0)),
            pl.BlockSpec((1, H), lambda i: (0, 0)),
            pl.BlockSpec((1, H), lambda i: (0, 0)),
        ],
        out_specs=[
            pl.BlockSpec((NQ, RPRE, HQ), lambda i: (0, i, 0)),
            pl.BlockSpec((RPRE, 1), lambda i: (i, 0)),
            pl.BlockSpec((RPRE, 1), lambda i: (i, 0)),
        ],
        out_shape=[
            jax.ShapeDtypeStruct((NQ, N, HQ), jnp.float32),
            jax.ShapeDtypeStruct((N, 1), jnp.float32),
            jax.ShapeDtypeStruct((N, 1), jnp.float32),
        ],
    )(x, W, a_s.reshape(1, H), a_d.reshape(1, H))


def _mk_body(asv_ref, adv_ref, m_ref):
    m = _lrelu(jnp.max(asv_ref[...]) + jnp.max(adv_ref[...]))
    m_ref[...] = jnp.full((8, 128), m, jnp.float32)


def _mk(asv, adv):
    return pl.pallas_call(
        _mk_body,
        out_shape=jax.ShapeDtypeStruct((8, 128), jnp.float32),
    )(asv, adv)


def _sc_body(h4_hbm, srcp_hbm, dstp_hbm, asv_hbm, adv_hbm, m_hbm,
             acc_hbm, pden_hbm,
             src_t, dst_t, w_t, asv_t, adv_t, pden_t, m_t,
             rb0, rb1, rb2, zbuf, accspm,
             sem0, sem1, sem2):
    c = lax.axis_index("c")
    s = lax.axis_index("s")

    pltpu.sync_copy(m_hbm.at[0, pl.ds(0, 16)], m_t)
    pltpu.sync_copy(asv_hbm, asv_t)
    pltpu.sync_copy(adv_hbm, adv_t)
    pltpu.sync_copy(srcp_hbm.at[s], src_t)
    pltpu.sync_copy(dstp_hbm.at[s], dst_t)

    @pl.loop(0, RFLUSH)
    def _(r):
        for f in range(0, HQ, 16):
            zbuf[r, pl.ds(f, 16)] = jnp.zeros((16,), jnp.float32)

    @pl.loop(0, N, step=16)
    def _(i):
        pden_t[pl.ds(i, 16)] = jnp.zeros((16,), jnp.float32)

    m16 = m_t[...]

    def _scale(buf, j, compute_w):
        @pl.loop(0, CH, step=16)
        def _(k):
            if compute_w:
                s16 = src_t[j, pl.ds(k, 16)]
                d16 = dst_t[j, pl.ds(k, 16)]
                e = (plsc.load_gather(asv_t, [s16])
                     + plsc.load_gather(adv_t, [d16]))
                w16 = jnp.exp(_lrelu(e) - m16)
                g = s * EPT + j * CH + k + lax.iota(jnp.int32, 16)
                w16 = jnp.where(g < E, w16, 0.0)
                w_t[j, pl.ds(k, 16)] = w16
                plsc.addupdate_scatter(pden_t, [d16], w16)
            else:
                w16 = w_t[j, pl.ds(k, 16)]
            for l in range(16):
                av = jnp.full((16,), w16[l], jnp.float32)
                for f in range(0, HQ, 16):
                    buf[k + l, pl.ds(f, 16)] = buf[k + l, pl.ds(f, 16)] * av

    for p in range(PPC):
        q = c * PPC + p
        hslab = h4_hbm.at[q]

        @pl.loop(0, RPT // RFLUSH)
        def _(k):
            pltpu.sync_copy(zbuf,
                            accspm.at[pl.ds(s * RPT + k * RFLUSH, RFLUSH)])

        plsc.subcore_barrier()

        bufs = (rb0, rb1, rb2)
        sems = (sem0, sem1, sem2)

        pltpu.async_copy(hslab.at[src_t.at[0]], bufs[0], sems[0])
        pltpu.async_copy(hslab.at[src_t.at[1]], bufs[1], sems[1])

        @pl.loop(2, NCHUNK + 2)
        def _(j):
            for par in range(3):
                @pl.when(lax.rem(j, 3) == par)
                def _():
                    new, old = bufs[par], bufs[(par + 1) % 3]
                    @pl.when(j >= 3)
                    def _():
                        pltpu.make_async_copy(
                            acc_hbm.at[q, pl.ds(0, CH)], new,
                            sems[par]).wait()

                    @pl.when(j < NCHUNK)
                    def _():
                        pltpu.async_copy(hslab.at[src_t.at[j]], new,
                                         sems[par])
                    pltpu.make_async_copy(
                        hslab.at[pl.ds(0, CH)], old,
                        sems[(par + 1) % 3]).wait()
                    _scale(old, j - 2, compute_w=(p == 0))
                    pltpu.async_copy(old, accspm.at[dst_t.at[j - 2]],
                                     sems[(par + 1) % 3], add=True)

        pltpu.make_async_copy(acc_hbm.at[q, pl.ds(0, CH)], bufs[0],
                              sems[0]).wait()

        if p == 0:
            @pl.when(c == 0)
            def _():
                pltpu.sync_copy(pden_t, pden_hbm.at[s])

        plsc.subcore_barrier()

        @pl.loop(0, RPT // RFLUSH)
        def _(k):
            base = s * RPT + k * RFLUSH
            pltpu.sync_copy(accspm.at[pl.ds(base, RFLUSH)],
                            acc_hbm.at[q, pl.ds(base, RFLUSH)])


def _sc_edges(h4, srcp, dstp, asv, adv, m):
    mesh = plsc.VectorSubcoreMesh(core_axis_name="c", subcore_axis_name="s")
    kern = pl.kernel(
        _sc_body,
        mesh=mesh,
        compiler_params=pltpu.CompilerParams(use_tc_tiling_on_sc=False,
                                             needs_layout_passes=False),
        out_type=[
            jax.ShapeDtypeStruct((NQ, N, HQ), jnp.float32),
            jax.ShapeDtypeStruct((NT, N), jnp.float32),
        ],
        scratch_types=[
            pltpu.VMEM((NCHUNK, CH), jnp.int32),
            pltpu.VMEM((NCHUNK, CH), jnp.int32),
            pltpu.VMEM((NCHUNK, CH), jnp.float32),
            pltpu.VMEM((N,), jnp.float32),
            pltpu.VMEM((N,), jnp.float32),
            pltpu.VMEM((N,), jnp.float32),
            pltpu.VMEM((16,), jnp.float32),
            pltpu.VMEM((CH, HQ), jnp.float32),
            pltpu.VMEM((CH, HQ), jnp.float32),
            pltpu.VMEM((CH, HQ), jnp.float32),
            pltpu.VMEM((RFLUSH, HQ), jnp.float32),
            pltpu.VMEM_SHARED((N, HQ), jnp.float32),
            pltpu.SemaphoreType.DMA,
            pltpu.SemaphoreType.DMA,
            pltpu.SemaphoreType.DMA,
        ],
    )
    return kern(h4, srcp, dstp, asv, adv, m)


def _den_body(pden_ref, den_ref):
    ones = jnp.ones((NT, 1), jnp.float32)
    den_ref[...] = lax.dot_general(pden_ref[...], ones,
                                   (((0,), (0,)), ((), ())),
                                   precision=lax.Precision.HIGHEST,
                                   preferred_element_type=jnp.float32)


def _den(pden):
    return pl.pallas_call(
        _den_body,
        out_shape=jax.ShapeDtypeStruct((N, 1), jnp.float32),
    )(pden)


def _post_body(relu, acc_ref, den_ref, h4_ref, asv_ref, adv_ref, m_ref,
               b_ref, out_ref):
    sw = jnp.exp(_lrelu(asv_ref[...] + adv_ref[...]) - m_ref[0:1, 0:1])
    den = den_ref[...] + sw
    cols = [acc_ref[q] + sw * h4_ref[q] for q in range(NQ)]
    o = jnp.concatenate(cols, axis=1) / den + b_ref[...]
    if relu:
        o = jnp.maximum(o, 0.0)
    out_ref[...] = o


def _post(acc, den, h4, asv, adv, m, b, relu):
    return pl.pallas_call(
        functools.partial(_post_body, relu),
        grid=(N // RPRE,),
        in_specs=[
            pl.BlockSpec((NQ, RPRE, HQ), lambda i: (0, i, 0)),
            pl.BlockSpec((RPRE, 1), lambda i: (i, 0)),
            pl.BlockSpec((NQ, RPRE, HQ), lambda i: (0, i, 0)),
            pl.BlockSpec((RPRE, 1), lambda i: (i, 0)),
            pl.BlockSpec((RPRE, 1), lambda i: (i, 0)),
            pl.BlockSpec((8, 128), lambda i: (0, 0)),
            pl.BlockSpec((1, H), lambda i: (0, 0)),
        ],
        out_specs=pl.BlockSpec((RPRE, H), lambda i: (i, 0)),
        out_shape=jax.ShapeDtypeStruct((N, H), jnp.float32),
    )(acc, den, h4, asv, adv, m, b.reshape(1, H))


def _postpre_body(acc_ref, den_ref, h4_ref, asv_ref, adv_ref, m_ref, b_ref,
                  w2_ref, as2_ref, ad2_ref, h4o_ref, asv2_ref, adv2_ref):
    sw = jnp.exp(_lrelu(asv_ref[...] + adv_ref[...]) - m_ref[0:1, 0:1])
    den = den_ref[...] + sw
    cols = [acc_ref[qq] + sw * h4_ref[qq] for qq in range(NQ)]
    o = jnp.concatenate(cols, axis=1) / den + b_ref[...]
    o = jnp.maximum(o, 0.0)
    h = jnp.dot(o, w2_ref[...], preferred_element_type=jnp.float32)
    for qq in range(NQ):
        h4o_ref[qq] = h[:, qq * HQ:(qq + 1) * HQ]
    asv2_ref[...] = jnp.sum(h * as2_ref[...], axis=1, keepdims=True)
    adv2_ref[...] = jnp.sum(h * ad2_ref[...], axis=1, keepdims=True)


def _postpre(acc, den, h4, asv, adv, m, b, W2, a_s2, a_d2):
    return pl.pallas_call(
        _postpre_body,
        grid=(N // RPRE,),
        in_specs=[
            pl.BlockSpec((NQ, RPRE, HQ), lambda i: (0, i, 0)),
            pl.BlockSpec((RPRE, 1), lambda i: (i, 0)),
            pl.BlockSpec((NQ, RPRE, HQ), lambda i: (0, i, 0)),
            pl.BlockSpec((RPRE, 1), lambda i: (i, 0)),
            pl.BlockSpec((RPRE, 1), lambda i: (i, 0)),
            pl.BlockSpec((8, 128), lambda i: (0, 0)),
            pl.BlockSpec((1, H), lambda i: (0, 0)),
            pl.BlockSpec((H, H), lambda i: (0, 0)),
            pl.BlockSpec((1, H), lambda i: (0, 0)),
            pl.BlockSpec((1, H), lambda i: (0, 0)),
        ],
        out_specs=[
            pl.BlockSpec((NQ, RPRE, HQ), lambda i: (0, i, 0)),
            pl.BlockSpec((RPRE, 1), lambda i: (i, 0)),
            pl.BlockSpec((RPRE, 1), lambda i: (i, 0)),
        ],
        out_shape=[
            jax.ShapeDtypeStruct((NQ, N, HQ), jnp.float32),
            jax.ShapeDtypeStruct((N, 1), jnp.float32),
            jax.ShapeDtypeStruct((N, 1), jnp.float32),
        ],
    )(acc, den, h4, asv, adv, m, b.reshape(1, H), W2,
      a_s2.reshape(1, H), a_d2.reshape(1, H))


def kernel(x, edge_index, W1, a_src1, a_dst1, b1, W2, a_src2, a_dst2, b2):
    src = edge_index[0].astype(jnp.int32)
    dst = edge_index[1].astype(jnp.int32)
    srcp = jnp.pad(src, (0, EPAD - E)).reshape(NT, NCHUNK, CH)
    dstp = jnp.pad(dst, (0, EPAD - E)).reshape(NT, NCHUNK, CH)

    h41, asv1, adv1 = _pre(x, W1, a_src1, a_dst1)
    m1 = _mk(asv1, adv1)
    acc1, pden1 = _sc_edges(h41, srcp, dstp,
                            asv1.reshape(N), adv1.reshape(N), m1)
    h42, asv2, adv2 = _postpre(acc1, _den(pden1), h41, asv1, adv1, m1, b1,
                               W2, a_src2, a_dst2)
    m2 = _mk(asv2, adv2)
    acc2, pden2 = _sc_edges(h42, srcp, dstp,
                            asv2.reshape(N), adv2.reshape(N), m2)
    return _post(acc2, _den(pden2), h42, asv2, adv2, m2, b2, relu=False)


# speedup vs baseline: 1.2258x; 1.0087x over previous
import functools

import jax
import jax.numpy as jnp
from jax import lax
from jax.experimental import pallas as pl
from jax.experimental.pallas import tpu as pltpu
from jax.experimental.pallas import tpu_sc as plsc

N = 10000
H = 128
HQ = 32
NQ = 4
E = 320000
NT = 16
NC = 2
PPC = NQ // NC
CH = 128
EPT = 20480
NCHUNK = EPT // CH
EPAD = NT * EPT
RPRE = 400
RFLUSH = 125
RPT = N // NT


def _lrelu(v):
    return jnp.where(v >= 0, v, 0.2 * v)


def _pre_body(x_ref, w_ref, as_ref, ad_ref, h4_ref, asv_ref, adv_ref):
    h = jnp.dot(x_ref[...], w_ref[...], preferred_element_type=jnp.float32)
    for q in range(NQ):
        h4_ref[q] = h[:, q * HQ:(q + 1) * HQ]
    asv_ref[...] = jnp.sum(h * as_ref[...], axis=1, keepdims=True)
    adv_ref[...] = jnp.sum(h * ad_ref[...], axis=1, keepdims=True)


def _pre(x, W, a_s, a_d):
    return pl.pallas_call(
        _pre_body,
        grid=(N // RPRE,),
        in_specs=[
            pl.BlockSpec((RPRE, H), lambda i: (i, 0)),
            pl.BlockSpec((H, H), lambda i: (0, 0)),
            pl.BlockSpec((1, H), lambda i: (0, 0)),
            pl.BlockSpec((1, H), lambda i: (0, 0)),
        ],
        out_specs=[
            pl.BlockSpec((NQ, RPRE, HQ), lambda i: (0, i, 0)),
            pl.BlockSpec((RPRE, 1), lambda i: (i, 0)),
            pl.BlockSpec((RPRE, 1), lambda i: (i, 0)),
        ],
        out_shape=[
            jax.ShapeDtypeStruct((NQ, N, HQ), jnp.float32),
            jax.ShapeDtypeStruct((N, 1), jnp.float32),
            jax.ShapeDtypeStruct((N, 1), jnp.float32),
        ],
    )(x, W, a_s.reshape(1, H), a_d.reshape(1, H))


def _mk_body(asv_ref, adv_ref, m_ref):
    m = _lrelu(jnp.max(asv_ref[...]) + jnp.max(adv_ref[...]))
    m_ref[...] = jnp.full((8, 128), m, jnp.float32)


def _mk(asv, adv):
    return pl.pallas_call(
        _mk_body,
        out_shape=jax.ShapeDtypeStruct((8, 128), jnp.float32),
    )(asv, adv)


def _sc_body(h4_hbm, srcp_hbm, dstp_hbm, asv_hbm, adv_hbm, m_hbm,
             acc_hbm, pden_hbm,
             src_t, dst_t, w_t, asv_t, adv_t, pden_t, m_t,
             rb0, rb1, rb2, zbuf, accspm,
             sem0, sem1, sem2):
    c = lax.axis_index("c")
    s = lax.axis_index("s")

    pltpu.sync_copy(m_hbm.at[0, pl.ds(0, 16)], m_t)
    pltpu.sync_copy(asv_hbm, asv_t)
    pltpu.sync_copy(adv_hbm, adv_t)
    pltpu.sync_copy(srcp_hbm.at[s], src_t)
    pltpu.sync_copy(dstp_hbm.at[s], dst_t)

    @pl.loop(0, RFLUSH)
    def _(r):
        for f in range(0, HQ, 16):
            zbuf[r, pl.ds(f, 16)] = jnp.zeros((16,), jnp.float32)

    @pl.loop(0, N, step=16)
    def _(i):
        pden_t[pl.ds(i, 16)] = jnp.zeros((16,), jnp.float32)

    m16 = m_t[...]

    def _scale(buf, j, compute_w):
        @pl.loop(0, CH, step=16)
        def _(k):
            if compute_w:
                s16 = src_t[j, pl.ds(k, 16)]
                d16 = dst_t[j, pl.ds(k, 16)]
                e = (plsc.load_gather(asv_t, [s16])
                     + plsc.load_gather(adv_t, [d16]))
                w16 = jnp.exp(_lrelu(e) - m16)
                g = s * EPT + j * CH + k + lax.iota(jnp.int32, 16)
                w16 = jnp.where(g < E, w16, 0.0)
                w_t[j, pl.ds(k, 16)] = w16
                plsc.addupdate_scatter(pden_t, [d16], w16)
            else:
                w16 = w_t[j, pl.ds(k, 16)]
            for l in range(16):
                av = jnp.full((16,), w16[l], jnp.float32)
                for f in range(0, HQ, 16):
                    buf[k + l, pl.ds(f, 16)] = buf[k + l, pl.ds(f, 16)] * av

    for p in range(PPC):
        q = c * PPC + p
        hslab = h4_hbm.at[q]

        @pl.loop(0, RPT // RFLUSH)
        def _(k):
            pltpu.sync_copy(zbuf,
                            accspm.at[pl.ds(s * RPT + k * RFLUSH, RFLUSH)])

        plsc.subcore_barrier()

        bufs = (rb0, rb1, rb2)
        sems = (sem0, sem1, sem2)

        def _gather(j, buf, sem):
            pltpu.async_copy(hslab.at[src_t.at[j, pl.ds(0, CH // 2)]],
                             buf.at[pl.ds(0, CH // 2)], sem)
            pltpu.async_copy(hslab.at[src_t.at[j, pl.ds(CH // 2, CH // 2)]],
                             buf.at[pl.ds(CH // 2, CH // 2)], sem)

        _gather(0, bufs[0], sems[0])
        _gather(1, bufs[1], sems[1])

        @pl.loop(2, NCHUNK + 2)
        def _(j):
            for par in range(3):
                @pl.when(lax.rem(j, 3) == par)
                def _():
                    new, old = bufs[par], bufs[(par + 1) % 3]
                    @pl.when(j >= 3)
                    def _():
                        pltpu.make_async_copy(
                            acc_hbm.at[q, pl.ds(0, CH)], new,
                            sems[par]).wait()

                    @pl.when(j < NCHUNK)
                    def _():
                        _gather(j, new, sems[par])
                    pltpu.make_async_copy(
                        hslab.at[pl.ds(0, CH)], old,
                        sems[(par + 1) % 3]).wait()
                    _scale(old, j - 2, compute_w=(p == 0))
                    pltpu.async_copy(old, accspm.at[dst_t.at[j - 2]],
                                     sems[(par + 1) % 3], add=True)

        pltpu.make_async_copy(acc_hbm.at[q, pl.ds(0, CH)], bufs[0],
                              sems[0]).wait()

        if p == 0:
            @pl.when(c == 0)
            def _():
                pltpu.sync_copy(pden_t, pden_hbm.at[s])

        plsc.subcore_barrier()

        @pl.loop(0, RPT // RFLUSH)
        def _(k):
            base = s * RPT + k * RFLUSH
            pltpu.sync_copy(accspm.at[pl.ds(base, RFLUSH)],
                            acc_hbm.at[q, pl.ds(base, RFLUSH)])


def _sc_edges(h4, srcp, dstp, asv, adv, m):
    mesh = plsc.VectorSubcoreMesh(core_axis_name="c", subcore_axis_name="s")
    kern = pl.kernel(
        _sc_body,
        mesh=mesh,
        compiler_params=pltpu.CompilerParams(use_tc_tiling_on_sc=False,
                                             needs_layout_passes=False),
        out_type=[
            jax.ShapeDtypeStruct((NQ, N, HQ), jnp.float32),
            jax.ShapeDtypeStruct((NT, N), jnp.float32),
        ],
        scratch_types=[
            pltpu.VMEM((NCHUNK, CH), jnp.int32),
            pltpu.VMEM((NCHUNK, CH), jnp.int32),
            pltpu.VMEM((NCHUNK, CH), jnp.float32),
            pltpu.VMEM((N,), jnp.float32),
            pltpu.VMEM((N,), jnp.float32),
            pltpu.VMEM((N,), jnp.float32),
            pltpu.VMEM((16,), jnp.float32),
            pltpu.VMEM((CH, HQ), jnp.float32),
            pltpu.VMEM((CH, HQ), jnp.float32),
            pltpu.VMEM((CH, HQ), jnp.float32),
            pltpu.VMEM((RFLUSH, HQ), jnp.float32),
            pltpu.VMEM_SHARED((N, HQ), jnp.float32),
            pltpu.SemaphoreType.DMA,
            pltpu.SemaphoreType.DMA,
            pltpu.SemaphoreType.DMA,
        ],
    )
    return kern(h4, srcp, dstp, asv, adv, m)


def _den_body(pden_ref, den_ref):
    ones = jnp.ones((NT, 1), jnp.float32)
    den_ref[...] = lax.dot_general(pden_ref[...], ones,
                                   (((0,), (0,)), ((), ())),
                                   precision=lax.Precision.HIGHEST,
                                   preferred_element_type=jnp.float32)


def _den(pden):
    return pl.pallas_call(
        _den_body,
        out_shape=jax.ShapeDtypeStruct((N, 1), jnp.float32),
    )(pden)


def _post_body(relu, acc_ref, den_ref, h4_ref, asv_ref, adv_ref, m_ref,
               b_ref, out_ref):
    sw = jnp.exp(_lrelu(asv_ref[...] + adv_ref[...]) - m_ref[0:1, 0:1])
    den = den_ref[...] + sw
    cols = [acc_ref[q] + sw * h4_ref[q] for q in range(NQ)]
    o = jnp.concatenate(cols, axis=1) / den + b_ref[...]
    if relu:
        o = jnp.maximum(o, 0.0)
    out_ref[...] = o


def _post(acc, den, h4, asv, adv, m, b, relu):
    return pl.pallas_call(
        functools.partial(_post_body, relu),
        grid=(N // RPRE,),
        in_specs=[
            pl.BlockSpec((NQ, RPRE, HQ), lambda i: (0, i, 0)),
            pl.BlockSpec((RPRE, 1), lambda i: (i, 0)),
            pl.BlockSpec((NQ, RPRE, HQ), lambda i: (0, i, 0)),
            pl.BlockSpec((RPRE, 1), lambda i: (i, 0)),
            pl.BlockSpec((RPRE, 1), lambda i: (i, 0)),
            pl.BlockSpec((8, 128), lambda i: (0, 0)),
            pl.BlockSpec((1, H), lambda i: (0, 0)),
        ],
        out_specs=pl.BlockSpec((RPRE, H), lambda i: (i, 0)),
        out_shape=jax.ShapeDtypeStruct((N, H), jnp.float32),
    )(acc, den, h4, asv, adv, m, b.reshape(1, H))


def _postpre_body(acc_ref, den_ref, h4_ref, asv_ref, adv_ref, m_ref, b_ref,
                  w2_ref, as2_ref, ad2_ref, h4o_ref, asv2_ref, adv2_ref):
    sw = jnp.exp(_lrelu(asv_ref[...] + adv_ref[...]) - m_ref[0:1, 0:1])
    den = den_ref[...] + sw
    cols = [acc_ref[qq] + sw * h4_ref[qq] for qq in range(NQ)]
    o = jnp.concatenate(cols, axis=1) / den + b_ref[...]
    o = jnp.maximum(o, 0.0)
    h = jnp.dot(o, w2_ref[...], preferred_element_type=jnp.float32)
    for qq in range(NQ):
        h4o_ref[qq] = h[:, qq * HQ:(qq + 1) * HQ]
    asv2_ref[...] = jnp.sum(h * as2_ref[...], axis=1, keepdims=True)
    adv2_ref[...] = jnp.sum(h * ad2_ref[...], axis=1, keepdims=True)


def _postpre(acc, den, h4, asv, adv, m, b, W2, a_s2, a_d2):
    return pl.pallas_call(
        _postpre_body,
        grid=(N // RPRE,),
        in_specs=[
            pl.BlockSpec((NQ, RPRE, HQ), lambda i: (0, i, 0)),
            pl.BlockSpec((RPRE, 1), lambda i: (i, 0)),
            pl.BlockSpec((NQ, RPRE, HQ), lambda i: (0, i, 0)),
            pl.BlockSpec((RPRE, 1), lambda i: (i, 0)),
            pl.BlockSpec((RPRE, 1), lambda i: (i, 0)),
            pl.BlockSpec((8, 128), lambda i: (0, 0)),
            pl.BlockSpec((1, H), lambda i: (0, 0)),
            pl.BlockSpec((H, H), lambda i: (0, 0)),
            pl.BlockSpec((1, H), lambda i: (0, 0)),
            pl.BlockSpec((1, H), lambda i: (0, 0)),
        ],
        out_specs=[
            pl.BlockSpec((NQ, RPRE, HQ), lambda i: (0, i, 0)),
            pl.BlockSpec((RPRE, 1), lambda i: (i, 0)),
            pl.BlockSpec((RPRE, 1), lambda i: (i, 0)),
        ],
        out_shape=[
            jax.ShapeDtypeStruct((NQ, N, HQ), jnp.float32),
            jax.ShapeDtypeStruct((N, 1), jnp.float32),
            jax.ShapeDtypeStruct((N, 1), jnp.float32),
        ],
    )(acc, den, h4, asv, adv, m, b.reshape(1, H), W2,
      a_s2.reshape(1, H), a_d2.reshape(1, H))


def kernel(x, edge_index, W1, a_src1, a_dst1, b1, W2, a_src2, a_dst2, b2):
    src = edge_index[0].astype(jnp.int32)
    dst = edge_index[1].astype(jnp.int32)
    srcp = jnp.pad(src, (0, EPAD - E)).reshape(NT, NCHUNK, CH)
    dstp = jnp.pad(dst, (0, EPAD - E)).reshape(NT, NCHUNK, CH)

    h41, asv1, adv1 = _pre(x, W1, a_src1, a_dst1)
    m1 = _mk(asv1, adv1)
    acc1, pden1 = _sc_edges(h41, srcp, dstp,
                            asv1.reshape(N), adv1.reshape(N), m1)
    h42, asv2, adv2 = _postpre(acc1, _den(pden1), h41, asv1, adv1, m1, b1,
                               W2, a_src2, a_dst2)
    m2 = _mk(asv2, adv2)
    acc2, pden2 = _sc_edges(h42, srcp, dstp,
                            asv2.reshape(N), adv2.reshape(N), m2)
    return _post(acc2, _den(pden2), h42, asv2, adv2, m2, b2, relu=False)
